# Initial kernel scaffold; baseline (speedup 1.0000x reference)
import jax
import jax.numpy as jnp
from jax.experimental import pallas as pl


def kernel(x, table):
    raise NotImplementedError("write your pallas kernel here")


# speedup vs baseline: 3.1697x; 3.1697x over previous
import functools

import jax
import jax.numpy as jnp
from jax import lax
from jax.experimental import pallas as pl
from jax.experimental.pallas import tpu as pltpu
from jax.experimental.pallas import tpu_sc as plsc

_VOCAB = 100000
_D = 128
_B = 4096 * 50
_NW = 32
_B_PER_W = _B // _NW
_CHUNK = 400
_NCHUNK = _B_PER_W // _CHUNK


def _make_gather():
    mesh = plsc.VectorSubcoreMesh(core_axis_name="c", subcore_axis_name="s")

    @functools.partial(
        pl.kernel,
        mesh=mesh,
        out_type=jax.ShapeDtypeStruct((_B, _D), jnp.float32),
        scratch_types=[
            pltpu.VMEM((_CHUNK,), jnp.int32),
            pltpu.VMEM((_CHUNK, _D), jnp.float32),
            pltpu.SemaphoreType.DMA,
        ],
    )
    def gather_kernel(idx_hbm, table_hbm, out_hbm, idx_v, rows_v, sem):
        wid = lax.axis_index("s") * 2 + lax.axis_index("c")
        base = wid * _B_PER_W

        def body(ci, carry):
            cbase = base + ci * _CHUNK
            pltpu.sync_copy(idx_hbm.at[pl.ds(cbase, _CHUNK)], idx_v)
            pltpu.async_copy(table_hbm.at[idx_v], rows_v, sem).wait()
            pltpu.sync_copy(rows_v, out_hbm.at[pl.ds(cbase, _CHUNK)])
            return carry

        lax.fori_loop(0, _NCHUNK, body, 0)

    return gather_kernel


_gather = _make_gather()


@jax.jit
def kernel(x, table):
    idx = x.reshape(-1).astype(jnp.int32)
    out = _gather(idx, table)
    return out.reshape(x.shape + (_D,))


# speedup vs baseline: 3.2708x; 1.0319x over previous
import functools

import jax
import jax.numpy as jnp
from jax import lax
from jax.experimental import pallas as pl
from jax.experimental.pallas import tpu as pltpu
from jax.experimental.pallas import tpu_sc as plsc

_VOCAB = 100000
_D = 128
_B = 4096 * 50
_NW = 32
_B_PER_W = _B // _NW
_CHUNK = 400
_NCHUNK = _B_PER_W // _CHUNK


def _make_gather():
    mesh = plsc.VectorSubcoreMesh(core_axis_name="c", subcore_axis_name="s")

    @functools.partial(
        pl.kernel,
        mesh=mesh,
        out_type=jax.ShapeDtypeStruct((_B, _D), jnp.float32),
        scratch_types=[
            pltpu.VMEM((_CHUNK,), jnp.int32),
            pltpu.VMEM((_CHUNK,), jnp.int32),
            pltpu.VMEM((_CHUNK, _D), jnp.float32),
            pltpu.VMEM((_CHUNK, _D), jnp.float32),
            pltpu.SemaphoreType.DMA,
            pltpu.SemaphoreType.DMA,
            pltpu.SemaphoreType.DMA,
            pltpu.SemaphoreType.DMA,
        ],
    )
    def gather_kernel(idx_hbm, table_hbm, out_hbm, idx_v0, idx_v1,
                      rows_v0, rows_v1, gsem0, gsem1, wsem0, wsem1):
        wid = lax.axis_index("s") * 2 + lax.axis_index("c")
        base = wid * _B_PER_W

        idx_v = (idx_v0, idx_v1)
        rows_v = (rows_v0, rows_v1)
        gsem = (gsem0, gsem1)
        wsem = (wsem0, wsem1)

        wdesc = [None, None]
        for ci in range(_NCHUNK):
            p = ci % 2
            if wdesc[p] is not None:
                wdesc[p].wait()
            cbase = base + ci * _CHUNK
            pltpu.sync_copy(idx_hbm.at[pl.ds(cbase, _CHUNK)], idx_v[p])
            pltpu.async_copy(table_hbm.at[idx_v[p]], rows_v[p], gsem[p]).wait()
            wdesc[p] = pltpu.async_copy(
                rows_v[p], out_hbm.at[pl.ds(cbase, _CHUNK)], wsem[p])
        wdesc[0].wait()
        wdesc[1].wait()

    return gather_kernel


_gather = _make_gather()


@jax.jit
def kernel(x, table):
    idx = x.reshape(-1).astype(jnp.int32)
    out = _gather(idx, table)
    return out.reshape(x.shape + (_D,))


# speedup vs baseline: 3.3235x; 1.0161x over previous
import functools

import jax
import jax.numpy as jnp
from jax import lax
from jax.experimental import pallas as pl
from jax.experimental.pallas import tpu as pltpu
from jax.experimental.pallas import tpu_sc as plsc

_VOCAB = 100000
_D = 128
_B = 4096 * 50
_NW = 32
_B_PER_W = _B // _NW
_CHUNK = 200
_NCHUNK = _B_PER_W // _CHUNK
_NBUF = 4


def _make_gather():
    mesh = plsc.VectorSubcoreMesh(core_axis_name="c", subcore_axis_name="s")

    @functools.partial(
        pl.kernel,
        mesh=mesh,
        out_type=jax.ShapeDtypeStruct((_B, _D), jnp.float32),
        scratch_types=(
            [pltpu.VMEM((_B_PER_W,), jnp.int32)]
            + [pltpu.VMEM((_CHUNK, _D), jnp.float32) for _ in range(_NBUF)]
            + [pltpu.SemaphoreType.DMA for _ in range(2 * _NBUF)]
        ),
    )
    def gather_kernel(idx_hbm, table_hbm, out_hbm, idx_v, *bufs):
        rows_v = bufs[:_NBUF]
        gsem = bufs[_NBUF:2 * _NBUF]
        wsem = bufs[2 * _NBUF:]
        wid = lax.axis_index("s") * 2 + lax.axis_index("c")
        base = wid * _B_PER_W

        pltpu.sync_copy(idx_hbm.at[pl.ds(base, _B_PER_W)], idx_v)

        gdesc = [None] * _NCHUNK
        wdesc = [None] * _NCHUNK

        def drain(ci):
            q = ci % _NBUF
            gdesc[ci].wait()
            wdesc[ci] = pltpu.async_copy(
                rows_v[q], out_hbm.at[pl.ds(base + ci * _CHUNK, _CHUNK)],
                wsem[q])

        for ci in range(_NCHUNK):
            p = ci % _NBUF
            if ci >= _NBUF:
                wdesc[ci - _NBUF].wait()
            gdesc[ci] = pltpu.async_copy(
                table_hbm.at[idx_v.at[pl.ds(ci * _CHUNK, _CHUNK)]],
                rows_v[p], gsem[p])
            if ci >= 2:
                drain(ci - 2)
        drain(_NCHUNK - 2)
        drain(_NCHUNK - 1)
        for ci in range(_NCHUNK - _NBUF, _NCHUNK):
            wdesc[ci].wait()

    return gather_kernel


_gather = _make_gather()


@jax.jit
def kernel(x, table):
    idx = x.reshape(-1).astype(jnp.int32)
    out = _gather(idx, table)
    return out.reshape(x.shape + (_D,))


# speedup vs baseline: 5.7989x; 1.7448x over previous
import functools

import jax
import jax.numpy as jnp
from jax import lax
from jax.experimental import pallas as pl
from jax.experimental.pallas import tpu as pltpu
from jax.experimental.pallas import tpu_sc as plsc

_VOCAB = 100000
_D = 128
_B = 4096 * 50
_NW = 32
_B_PER_W = _B // _NW
_CHUNK = 200
_NCHUNK = _B_PER_W // _CHUNK
_NBUF = 4


def _make_gather():
    mesh = plsc.VectorSubcoreMesh(core_axis_name="c", subcore_axis_name="s")

    hist = 50
    batch = _B // hist
    b_per_w = batch // _NW
    b_per_chunk = _CHUNK // hist

    @functools.partial(
        pl.kernel,
        mesh=mesh,
        out_type=jax.ShapeDtypeStruct((batch, hist, _D), jnp.float32),
        scratch_types=(
            [pltpu.VMEM((_B_PER_W,), jnp.int32)]
            + [pltpu.VMEM((_CHUNK, _D), jnp.float32) for _ in range(_NBUF)]
            + [pltpu.SemaphoreType.DMA for _ in range(2 * _NBUF)]
        ),
    )
    def gather_kernel(idx_hbm, table_hbm, out_hbm, idx_v, *bufs):
        rows_v = bufs[:_NBUF]
        gsem = bufs[_NBUF:2 * _NBUF]
        wsem = bufs[2 * _NBUF:]
        wid = lax.axis_index("s") * 2 + lax.axis_index("c")
        base = wid * _B_PER_W

        pltpu.sync_copy(idx_hbm.at[pl.ds(base, _B_PER_W)], idx_v)

        gdesc = [None] * _NCHUNK
        wdesc = [None] * _NCHUNK

        def drain(ci):
            q = ci % _NBUF
            gdesc[ci].wait()
            b0 = wid * b_per_w + ci * b_per_chunk
            for j in range(b_per_chunk):
                wdesc[ci] = pltpu.async_copy(
                    rows_v[q].at[pl.ds(j * hist, hist)],
                    out_hbm.at[b0 + j], wsem[q])

        def drain_writes(ci):
            q = ci % _NBUF
            for j in range(b_per_chunk):
                wdesc[ci].wait()

        for ci in range(_NCHUNK):
            p = ci % _NBUF
            if ci >= _NBUF:
                drain_writes(ci - _NBUF)
            gdesc[ci] = pltpu.async_copy(
                table_hbm.at[idx_v.at[pl.ds(ci * _CHUNK, _CHUNK)]],
                rows_v[p], gsem[p])
            if ci >= 2:
                drain(ci - 2)
        drain(_NCHUNK - 2)
        drain(_NCHUNK - 1)
        for ci in range(_NCHUNK - _NBUF, _NCHUNK):
            drain_writes(ci)

    return gather_kernel


_gather = _make_gather()


@jax.jit
def kernel(x, table):
    idx = x.reshape(-1).astype(jnp.int32)
    return _gather(idx, table)


# speedup vs baseline: 5.8107x; 1.0020x over previous
import functools

import jax
import jax.numpy as jnp
from jax import lax
from jax.experimental import pallas as pl
from jax.experimental.pallas import tpu as pltpu
from jax.experimental.pallas import tpu_sc as plsc

_VOCAB = 100000
_D = 128
_B = 4096 * 50
_NW = 32
_B_PER_W = _B // _NW
_CHUNK = 200
_NCHUNK = _B_PER_W // _CHUNK
_NBUF = 4


def _make_gather():
    mesh = plsc.VectorSubcoreMesh(core_axis_name="c", subcore_axis_name="s")

    hist = 50
    batch = _B // hist
    b_per_w = batch // _NW
    b_per_chunk = _CHUNK // hist

    @functools.partial(
        pl.kernel,
        mesh=mesh,
        compiler_params=pltpu.CompilerParams(use_tc_tiling_on_sc=True),
        out_type=jax.ShapeDtypeStruct((batch, hist, _D), jnp.float32),
        scratch_types=(
            [pltpu.VMEM((_B_PER_W,), jnp.int32)]
            + [pltpu.VMEM((_CHUNK, _D), jnp.float32) for _ in range(_NBUF)]
            + [pltpu.SemaphoreType.DMA for _ in range(2 * _NBUF)]
        ),
    )
    def gather_kernel(idx_hbm, table_hbm, out_hbm, idx_v, *bufs):
        rows_v = bufs[:_NBUF]
        gsem = bufs[_NBUF:2 * _NBUF]
        wsem = bufs[2 * _NBUF:]
        wid = lax.axis_index("s") * 2 + lax.axis_index("c")
        base = wid * _B_PER_W

        pltpu.sync_copy(idx_hbm.at[pl.ds(base, _B_PER_W)], idx_v)

        gdesc = [None] * _NCHUNK
        wdesc = [None] * _NCHUNK

        def drain(ci):
            q = ci % _NBUF
            gdesc[ci].wait()
            b0 = wid * b_per_w + ci * b_per_chunk
            for j in range(b_per_chunk):
                wdesc[ci] = pltpu.async_copy(
                    rows_v[q].at[pl.ds(j * hist, hist)],
                    out_hbm.at[b0 + j], wsem[q])

        def drain_writes(ci):
            q = ci % _NBUF
            for j in range(b_per_chunk):
                wdesc[ci].wait()

        for ci in range(_NCHUNK):
            p = ci % _NBUF
            if ci >= _NBUF:
                drain_writes(ci - _NBUF)
            gdesc[ci] = pltpu.async_copy(
                table_hbm.at[idx_v.at[pl.ds(ci * _CHUNK, _CHUNK)]],
                rows_v[p], gsem[p])
            if ci >= 2:
                drain(ci - 2)
        drain(_NCHUNK - 2)
        drain(_NCHUNK - 1)
        for ci in range(_NCHUNK - _NBUF, _NCHUNK):
            drain_writes(ci)

    return gather_kernel


_gather = _make_gather()


@jax.jit
def kernel(x, table):
    idx = x.reshape(-1).astype(jnp.int32)
    return _gather(idx, table)


# speedup vs baseline: 10.0625x; 1.7317x over previous
import functools

import jax
import jax.numpy as jnp
from jax import lax
from jax.experimental import pallas as pl
from jax.experimental.pallas import tpu as pltpu
from jax.experimental.pallas import tpu_sc as plsc

_VOCAB = 100000
_D = 128
_HIST = 50
_BATCH = 4096
_B = _BATCH * _HIST
_NW = 32
_B_PER_W = _B // _NW
_CHUNK = 200
_NCHUNK = _B_PER_W // _CHUNK
_NBUF = 4


def _make_gather():
    mesh = plsc.VectorSubcoreMesh(core_axis_name="c", subcore_axis_name="s")

    @functools.partial(
        pl.kernel,
        mesh=mesh,
        out_type=jax.ShapeDtypeStruct((_B, _D), jnp.float32),
        scratch_types=(
            [pltpu.VMEM((_B_PER_W,), jnp.int32)]
            + [pltpu.VMEM((_CHUNK, _D), jnp.float32) for _ in range(_NBUF)]
            + [pltpu.SemaphoreType.DMA for _ in range(2 * _NBUF)]
        ),
    )
    def gather_kernel(idx_hbm, table_hbm, out_hbm, idx_v, *bufs):
        rows_v = bufs[:_NBUF]
        gsem = bufs[_NBUF:2 * _NBUF]
        wsem = bufs[2 * _NBUF:]
        wid = lax.axis_index("s") * 2 + lax.axis_index("c")
        base = wid * _B_PER_W

        pltpu.sync_copy(idx_hbm.at[pl.ds(base, _B_PER_W)], idx_v)

        ngroup = _NCHUNK // _NBUF

        def group(j, carry):
            goff = j * _NBUF * _CHUNK

            def wait_writes():
                for b in range(_NBUF):
                    pltpu.make_async_copy(
                        rows_v[b],
                        out_hbm.at[pl.ds(base + goff + b * _CHUNK, _CHUNK)],
                        wsem[b]).wait()

            pl.when(j > 0)(wait_writes)
            for b in range(_NBUF):
                pltpu.async_copy(
                    table_hbm.at[idx_v.at[pl.ds(goff + b * _CHUNK, _CHUNK)]],
                    rows_v[b], gsem[b])
            for b in range(_NBUF):
                pltpu.make_async_copy(
                    table_hbm.at[idx_v.at[pl.ds(goff + b * _CHUNK, _CHUNK)]],
                    rows_v[b], gsem[b]).wait()
                pltpu.async_copy(
                    rows_v[b],
                    out_hbm.at[pl.ds(base + goff + b * _CHUNK, _CHUNK)],
                    wsem[b])
            return carry

        lax.fori_loop(0, ngroup, group, 0)
        for b in range(_NBUF):
            pltpu.make_async_copy(
                rows_v[b],
                out_hbm.at[pl.ds(base + (ngroup - 1) * _NBUF * _CHUNK
                                 + b * _CHUNK, _CHUNK)],
                wsem[b]).wait()

    return gather_kernel


_gather = _make_gather()


@jax.jit
def kernel(x, table):
    idx = jnp.transpose(x).reshape(-1).astype(jnp.int32)
    out = _gather(idx, table)
    return jnp.transpose(out.reshape(_HIST, _BATCH, _D), (1, 0, 2))


# speedup vs baseline: 10.2902x; 1.0226x over previous
import functools

import jax
import jax.numpy as jnp
from jax import lax
from jax.experimental import pallas as pl
from jax.experimental.pallas import tpu as pltpu
from jax.experimental.pallas import tpu_sc as plsc

_VOCAB = 100000
_D = 128
_HIST = 50
_BATCH = 4096
_B = _BATCH * _HIST
_NW = 32
_B_PER_W = _B // _NW
_CHUNK = 200
_NCHUNK = _B_PER_W // _CHUNK
_NBUF = 4


def _make_gather():
    mesh = plsc.VectorSubcoreMesh(core_axis_name="c", subcore_axis_name="s")

    @functools.partial(
        pl.kernel,
        mesh=mesh,
        out_type=jax.ShapeDtypeStruct((_B, _D), jnp.float32),
        scratch_types=(
            [pltpu.VMEM((_B_PER_W,), jnp.int32)]
            + [pltpu.VMEM((_CHUNK, _D), jnp.float32) for _ in range(_NBUF)]
            + [pltpu.SemaphoreType.DMA for _ in range(2 * _NBUF)]
        ),
    )
    def gather_kernel(idx_hbm, table_hbm, out_hbm, idx_v, *bufs):
        rows_v = bufs[:_NBUF]
        gsem = bufs[_NBUF:2 * _NBUF]
        wsem = bufs[2 * _NBUF:]
        wid = lax.axis_index("s") * 2 + lax.axis_index("c")
        base = wid * _B_PER_W

        pltpu.sync_copy(idx_hbm.at[pl.ds(base, _B_PER_W)], idx_v)

        gdesc = [None] * _NCHUNK
        wdesc = [None] * _NCHUNK

        def drain(ci):
            q = ci % _NBUF
            gdesc[ci].wait()
            wdesc[ci] = pltpu.async_copy(
                rows_v[q], out_hbm.at[pl.ds(base + ci * _CHUNK, _CHUNK)],
                wsem[q])

        for ci in range(_NCHUNK):
            p = ci % _NBUF
            if ci >= _NBUF:
                wdesc[ci - _NBUF].wait()
            gdesc[ci] = pltpu.async_copy(
                table_hbm.at[idx_v.at[pl.ds(ci * _CHUNK, _CHUNK)]],
                rows_v[p], gsem[p])
            if ci >= 3:
                drain(ci - 3)
        drain(_NCHUNK - 3)
        drain(_NCHUNK - 2)
        drain(_NCHUNK - 1)
        for ci in range(_NCHUNK - _NBUF, _NCHUNK):
            wdesc[ci].wait()

    return gather_kernel


_gather = _make_gather()


@jax.jit
def kernel(x, table):
    idx = jnp.transpose(x).reshape(-1).astype(jnp.int32)
    out = _gather(idx, table)
    return jnp.transpose(out.reshape(_HIST, _BATCH, _D), (1, 0, 2))
